# baseline (speedup 1.0000x reference)
import functools

import jax
import jax.numpy as jnp
from jax import lax
from jax.experimental import pallas as pl
from jax.experimental.pallas import tpu as pltpu
from jax.experimental.pallas import tpu_sc as plsc

_B = 4
_N = 4096
_K = 17
_R = 256
_NC = 2
_NS = 16
_NW = _NC * _NS
_NPW = (_B * _N) // _NW


def _tc_body(qr, kr, out_ref):
    qr = qr[0]
    kr = kr[0]

    dn = (((0,), (0,)), ((), ()))
    qk = jax.lax.dot_general(qr, kr, dn, precision=jax.lax.Precision.DEFAULT,
                             preferred_element_type=jnp.float32)
    q2 = jnp.sum(qr * qr, axis=0)
    r2 = jnp.sum(kr * kr, axis=0)
    d2 = q2[:, None] + r2[None, :] - 2.0 * qk

    ci = jax.lax.broadcasted_iota(jnp.int32, (_R, _N), 1)
    sb = jax.lax.bitcast_convert_type(d2, jnp.int32)
    cif = ci ^ (jax.lax.shift_right_arithmetic(sb, 31) & 0xFFF)
    keys = jax.lax.bitcast_convert_type((sb & ~0xFFF) | cif, jnp.float32)
    m0 = jnp.min(keys, axis=1, keepdims=True)

    kiT = jax.lax.broadcasted_iota(jnp.int32, (_K - 1, _R), 0)
    colsT = jnp.zeros((_K - 1, _R), jnp.int32)

    def _next(i, carry):
        m, colsT = carry
        m = jnp.min(jnp.where(keys > m, keys, jnp.inf), axis=1,
                    keepdims=True)
        mb = jax.lax.bitcast_convert_type(m, jnp.int32)
        col = (mb & 0xFFF) ^ (jax.lax.shift_right_arithmetic(mb, 31) & 0xFFF)
        colsT = jnp.where(kiT == i, jnp.reshape(col, (1, _R)), colsT)
        return m, colsT

    _, colsT = jax.lax.fori_loop(0, _K - 1, _next, (m0, colsT), unroll=True)
    out_ref[0] = colsT


def _nsqrt(s):
    i = jax.lax.bitcast_convert_type(s, jnp.int32)
    x = jax.lax.bitcast_convert_type(
        jax.lax.shift_right_arithmetic(i, 1) + 0x1FBD1DF6, jnp.float32)
    x = 0.5 * (x + s / x)
    x = 0.5 * (x + s / x)
    return x


def _sc_body(rxh, ryh, rzh, pxh, pyh, pzh, cols_hbm, out_hbm,
             rx, ry, rz, px, py, pz, cv, st, acc):
    wid = lax.axis_index("s") * _NC + lax.axis_index("c")
    b = wid // (_NW // _B)
    n0 = (wid % (_NW // _B)) * _NPW

    pltpu.sync_copy(rxh.at[pl.ds(b * _N, _N)], rx)
    pltpu.sync_copy(ryh.at[pl.ds(b * _N, _N)], ry)
    pltpu.sync_copy(rzh.at[pl.ds(b * _N, _N)], rz)
    pltpu.sync_copy(pxh.at[pl.ds(b * _N, _N)], px)
    pltpu.sync_copy(pyh.at[pl.ds(b * _N, _N)], py)
    pltpu.sync_copy(pzh.at[pl.ds(b * _N, _N)], pz)
    pltpu.sync_copy(cols_hbm.at[b, :, pl.ds(n0, _NPW)], cv)

    acc[...] = jnp.zeros((16,), jnp.float32)

    def _group(g, _):
        base = g * 16
        qx = rx[pl.ds(n0 + base, 16)]
        qy = ry[pl.ds(n0 + base, 16)]
        qz = rz[pl.ds(n0 + base, 16)]
        sx = px[pl.ds(n0 + base, 16)]
        sy = py[pl.ds(n0 + base, 16)]
        sz = pz[pl.ds(n0 + base, 16)]
        for k in range(_K - 1):
            idx = cv[k, pl.ds(base, 16)]
            gx = plsc.load_gather(rx, [idx])
            gy = plsc.load_gather(ry, [idx])
            gz = plsc.load_gather(rz, [idx])
            dx = gx - qx
            dy = gy - qy
            dz = gz - qz
            dd = dx * dx + dy * dy + dz * dz
            hx = plsc.load_gather(px, [idx])
            hy = plsc.load_gather(py, [idx])
            hz = plsc.load_gather(pz, [idx])
            ex = hx - sx
            ey = hy - sy
            ez = hz - sz
            ee = ex * ex + ey * ey + ez * ez
            acc[...] += jnp.abs(_nsqrt(dd) - _nsqrt(ee))
        return _

    lax.fori_loop(0, _NPW // 16, _group, 0)
    st[...] = acc[...]
    pltpu.sync_copy(st, out_hbm.at[wid])


@functools.cache
def _sc_kernel():
    return pl.kernel(
        _sc_body,
        out_type=jax.ShapeDtypeStruct((_NW, 16), jnp.float32),
        mesh=plsc.VectorSubcoreMesh(core_axis_name="c",
                                    subcore_axis_name="s"),
        compiler_params=pltpu.CompilerParams(needs_layout_passes=False),
        scratch_types=[
            pltpu.VMEM((_N,), jnp.float32),
            pltpu.VMEM((_N,), jnp.float32),
            pltpu.VMEM((_N,), jnp.float32),
            pltpu.VMEM((_N,), jnp.float32),
            pltpu.VMEM((_N,), jnp.float32),
            pltpu.VMEM((_N,), jnp.float32),
            pltpu.VMEM((_K - 1, _NPW), jnp.int32),
            pltpu.VMEM((16,), jnp.float32),
            pltpu.VMEM((16,), jnp.float32),
        ],
    )


def kernel(points_ref, points):
    ref_t = jnp.transpose(points_ref, (0, 2, 1))
    pts_t = jnp.transpose(points, (0, 2, 1))

    cols = pl.pallas_call(
        _tc_body,
        grid=(_B, _N // _R),
        in_specs=[
            pl.BlockSpec((1, 3, _R), lambda b, r: (b, 0, r)),
            pl.BlockSpec((1, 3, _N), lambda b, r: (b, 0, 0)),
        ],
        out_specs=pl.BlockSpec((1, _K - 1, _R), lambda b, r: (b, 0, r)),
        out_shape=jax.ShapeDtypeStruct((_B, _K - 1, _N), jnp.int32),
    )(ref_t, ref_t)

    partials = _sc_kernel()(
        ref_t[:, 0].reshape(-1), ref_t[:, 1].reshape(-1),
        ref_t[:, 2].reshape(-1), pts_t[:, 0].reshape(-1),
        pts_t[:, 1].reshape(-1), pts_t[:, 2].reshape(-1), cols)
    return jnp.sum(partials) * (1.0 / (_B * _N * (_K - 1)))
